# Initial kernel scaffold; baseline (speedup 1.0000x reference)
import jax
import jax.numpy as jnp
from jax.experimental import pallas as pl


def kernel(last_hidden_state, input_ids):
    raise NotImplementedError("write your pallas kernel here")


# speedup vs baseline: 9.7793x; 9.7793x over previous
import functools

import jax
import jax.numpy as jnp
from jax import lax
from jax.experimental import pallas as pl
from jax.experimental.pallas import tpu as pltpu
from jax.experimental.pallas import tpu_sc as plsc

B = 4
S = 8192
D = 2048
LANES = 16
SUB_PER_BATCH = 8
CHUNK = S // SUB_PER_BATCH
STEPS = CHUNK // LANES

_mesh = plsc.VectorSubcoreMesh(core_axis_name="c", subcore_axis_name="s")


@functools.partial(
    pl.kernel,
    mesh=_mesh,
    out_type=jax.ShapeDtypeStruct((B * D,), jnp.float32),
    scratch_types=[
        pltpu.VMEM((CHUNK,), jnp.int32),
        pltpu.VMEM((LANES,), jnp.int32),
        pltpu.VMEM_SHARED((16 * LANES,), jnp.int32),
        pltpu.VMEM((SUB_PER_BATCH * LANES,), jnp.int32),
        pltpu.VMEM((LANES,), jnp.int32),
        pltpu.VMEM((LANES, D), jnp.float32),
        pltpu.VMEM((D,), jnp.float32),
        pltpu.SemaphoreType.DMA,
    ],
)
def _clip_argmax_sc(hidden_hbm, ids_hbm, out_hbm,
                    ids_v, acc_v, shared_keys, part_v, idx_v,
                    rows_v, out_v, sem):
    c = lax.axis_index("c")
    s = lax.axis_index("s")
    b = c * 2 + s // SUB_PER_BATCH
    chunk = s % SUB_PER_BATCH
    base = chunk * CHUNK
    lane = lax.iota(jnp.int32, LANES)

    pltpu.sync_copy(ids_hbm.at[pl.ds(b * S + base, CHUNK)], ids_v)

    def step(j, acc):
        v = ids_v[pl.ds(j * LANES, LANES)]
        key = v * S + (S - 1 - base - j * LANES) - lane
        return jnp.maximum(acc, key)

    acc = lax.fori_loop(0, STEPS, step,
                        jnp.full((LANES,), -2**31, jnp.int32))
    acc_v[...] = acc
    pltpu.sync_copy(acc_v, shared_keys.at[pl.ds(s * LANES, LANES)])
    plsc.subcore_barrier()

    @pl.when(chunk == 0)
    def _leader():
        bc = s // SUB_PER_BATCH
        pltpu.sync_copy(
            shared_keys.at[pl.ds(bc * SUB_PER_BATCH * LANES,
                                 SUB_PER_BATCH * LANES)],
            part_v)
        m = part_v[pl.ds(0, LANES)]
        for i in range(1, SUB_PER_BATCH):
            m = jnp.maximum(m, part_v[pl.ds(i * LANES, LANES)])
        best = m[0]
        for i in range(1, LANES):
            best = jnp.maximum(best, m[i])
        idx = (S - 1) - lax.rem(best, S)
        row = b * S + idx
        idx_v[...] = jnp.full((LANES,), 0, jnp.int32) + row
        pltpu.async_copy(hidden_hbm.at[idx_v], rows_v, sem).wait()

        def pw(j, carry):
            x = rows_v[0, pl.ds(j * LANES, LANES)]
            x2 = x * x
            out_v[pl.ds(j * LANES, LANES)] = x2 * x2
            return carry

        lax.fori_loop(0, D // LANES, pw, 0)
        pltpu.sync_copy(out_v, out_hbm.at[pl.ds(b * D, D)])


def kernel(last_hidden_state, input_ids):
    ids = input_ids.astype(jnp.int32).reshape(B * S)
    hidden = last_hidden_state.reshape(B * S, D)
    return _clip_argmax_sc(hidden, ids).reshape(B, D)
